# Initial kernel scaffold; baseline (speedup 1.0000x reference)
import jax
import jax.numpy as jnp
from jax.experimental import pallas as pl


def kernel(memory, source_nodes, neighbors, edge_idxs, edge_deltas, edge_features, time_w, time_b, Wq, Wk, Wv, Wo, fc1_w, fc1_b, fc2_w, fc2_b):
    raise NotImplementedError("write your pallas kernel here")


# speedup vs baseline: 3.1775x; 3.1775x over previous
import functools
import math

import jax
import jax.numpy as jnp
from jax import lax
from jax.experimental import pallas as pl
from jax.experimental.pallas import tpu as pltpu
from jax.experimental.pallas import tpu_sc as plsc

N = 10000
D = 256
DT = 100
DE = 16
B = 4096
NN = 16
NE = 160000
H = 2
QD = D + DT
KD = D + DT + DE
DH = QD // H

BLK = 256


def _make_sc_gather():
  info = plsc.get_sparse_core_info()
  nc, ns = info.num_cores, info.num_subcores
  nw = nc * ns
  ngh_per_w = (B * NN) // nw
  ngh_chunk = 128
  n_chunks = ngh_per_w // ngh_chunk
  src_per_w = B // nw
  ef_per_w = (B * NN) // nw

  mesh = plsc.VectorSubcoreMesh(core_axis_name="c", subcore_axis_name="s")

  @functools.partial(
      pl.kernel,
      mesh=mesh,
      compiler_params=pltpu.CompilerParams(use_tc_tiling_on_sc=False),
      out_type=[
          jax.ShapeDtypeStruct((B * NN, D), jnp.float32),
          jax.ShapeDtypeStruct((B, D), jnp.float32),
          jax.ShapeDtypeStruct((B * NN, DE), jnp.float32),
      ],
      scratch_types=[
          pltpu.VMEM((ngh_chunk,), jnp.int32),
          pltpu.VMEM((ngh_chunk, D), jnp.float32),
          pltpu.VMEM((src_per_w,), jnp.int32),
          pltpu.VMEM((ef_per_w,), jnp.int32),
          pltpu.VMEM((ef_per_w, DE), jnp.float32),
          pltpu.SemaphoreType.DMA,
      ],
  )
  def sc_gather(mem_hbm, nbr_hbm, sidx_hbm, eidx_hbm, ef_hbm,
                ngh_out, src_out, ef_out,
                idx_v, rows_v, sidx_v, eidx_v, efrows_v, sem):
    wid = lax.axis_index("s") * nc + lax.axis_index("c")

    ebase = wid * ef_per_w
    pltpu.sync_copy(eidx_hbm.at[pl.ds(ebase, ef_per_w)], eidx_v)
    pltpu.async_copy(ef_hbm.at[eidx_v], efrows_v, sem).wait()
    pltpu.sync_copy(efrows_v, ef_out.at[pl.ds(ebase, ef_per_w)])

    sbase = wid * src_per_w
    pltpu.sync_copy(sidx_hbm.at[pl.ds(sbase, src_per_w)], sidx_v)
    pltpu.async_copy(mem_hbm.at[sidx_v], rows_v.at[pl.ds(0, src_per_w)],
                     sem).wait()
    pltpu.sync_copy(rows_v.at[pl.ds(0, src_per_w)],
                    src_out.at[pl.ds(sbase, src_per_w)])

    for j in range(n_chunks):
      base = wid * ngh_per_w + j * ngh_chunk
      pltpu.sync_copy(nbr_hbm.at[pl.ds(base, ngh_chunk)], idx_v)
      pltpu.async_copy(mem_hbm.at[idx_v], rows_v, sem).wait()
      pltpu.sync_copy(rows_v, ngh_out.at[pl.ds(base, ngh_chunk)])

  return sc_gather


def _tc_body(src_ref, ngh_ref, ef_ref, nbr_ref, dt_ref,
             bm0_ref, bm1_ref, bt0_ref, bt1_ref, be0_ref, be1_ref,
             cm0_ref, cm1_ref, ct0_ref, ct1_ref, ce0_ref, ce1_ref,
             vm0_ref, vm1_ref, vt0_ref, vt1_ref, ve0_ref, ve1_ref,
             fc1a_ref, fc1b_ref, fc1bias_ref, fc2_ref, fc2bias_ref,
             tw_ref, tb_ref, out_ref):
  f32 = jnp.float32
  src = src_ref[...]
  tw = tw_ref[...]
  tb = tb_ref[...]

  def mm(a, b):
    return jax.lax.dot(a, b, preferred_element_type=f32)

  um0 = mm(src, bm0_ref[...]) + cm0_ref[...]
  um1 = mm(src, bm1_ref[...]) + cm1_ref[...]
  ut0 = mm(src, bt0_ref[...]) + ct0_ref[...]
  ut1 = mm(src, bt1_ref[...]) + ct1_ref[...]
  ue0 = mm(src, be0_ref[...]) + ce0_ref[...]
  ue1 = mm(src, be1_ref[...]) + ce1_ref[...]

  def rsum(x):
    return jnp.sum(x, axis=1, keepdims=True)

  neg = jnp.float32(-1e10)
  s0l, s1l, ml = [], [], []
  for n in range(NN):
    ngh_n = ngh_ref[:, n * D:(n + 1) * D]
    ef_n = ef_ref[:, n, :]
    dt_n = dt_ref[:, n:n + 1]
    et_n = jnp.cos(dt_n * tw + tb)
    m_n = nbr_ref[:, n:n + 1] == 0
    s0 = rsum(ngh_n * um0) + rsum(et_n * ut0) + rsum(ef_n * ue0)
    s1 = rsum(ngh_n * um1) + rsum(et_n * ut1) + rsum(ef_n * ue1)
    s0l.append(jnp.where(m_n, neg, s0))
    s1l.append(jnp.where(m_n, neg, s1))
    ml.append(m_n)

  mx0 = functools.reduce(jnp.maximum, s0l)
  mx1 = functools.reduce(jnp.maximum, s1l)
  e0l = [jnp.exp(s - mx0) for s in s0l]
  e1l = [jnp.exp(s - mx1) for s in s1l]
  inv0 = 1.0 / functools.reduce(jnp.add, e0l)
  inv1 = 1.0 / functools.reduce(jnp.add, e1l)

  accm0 = jnp.zeros((BLK, D), f32)
  accm1 = jnp.zeros((BLK, D), f32)
  acct0 = jnp.zeros((BLK, DT), f32)
  acct1 = jnp.zeros((BLK, DT), f32)
  acce0 = jnp.zeros((BLK, DE), f32)
  acce1 = jnp.zeros((BLK, DE), f32)
  for n in range(NN):
    ngh_n = ngh_ref[:, n * D:(n + 1) * D]
    ef_n = ef_ref[:, n, :]
    dt_n = dt_ref[:, n:n + 1]
    et_n = jnp.cos(dt_n * tw + tb)
    w0 = e0l[n] * inv0
    w1 = e1l[n] * inv1
    accm0 = accm0 + w0 * ngh_n
    accm1 = accm1 + w1 * ngh_n
    acct0 = acct0 + w0 * et_n
    acct1 = acct1 + w1 * et_n
    acce0 = acce0 + w0 * ef_n
    acce1 = acce1 + w1 * ef_n

  attn_out = (mm(accm0, vm0_ref[...]) + mm(accm1, vm1_ref[...]) +
              mm(acct0, vt0_ref[...]) + mm(acct1, vt1_ref[...]) +
              mm(acce0, ve0_ref[...]) + mm(acce1, ve1_ref[...]))

  all_masked = functools.reduce(jnp.logical_and, ml)
  attn_out = jnp.where(all_masked, 0.0, attn_out)

  h1 = mm(attn_out, fc1a_ref[...]) + mm(src, fc1b_ref[...]) + fc1bias_ref[...]
  h1 = jnp.maximum(h1, 0.0)
  out_ref[...] = mm(h1, fc2_ref[...]) + fc2bias_ref[...]


def _tc_call(src_g, ngh2, ef3, neighbors, edge_deltas, weights):
  f32 = jnp.float32
  grid = (B // BLK,)
  bspec_batch2 = lambda w: pl.BlockSpec((BLK, w), lambda i: (i, 0))
  bconst2 = lambda r, c: pl.BlockSpec((r, c), lambda i: (0, 0))

  in_specs = [
      bspec_batch2(D),
      bspec_batch2(NN * D),
      pl.BlockSpec((BLK, NN, DE), lambda i: (i, 0, 0)),
      bspec_batch2(NN),
      bspec_batch2(NN),
      bconst2(D, D), bconst2(D, D),
      bconst2(D, DT), bconst2(D, DT),
      bconst2(D, DE), bconst2(D, DE),
      bconst2(1, D), bconst2(1, D),
      bconst2(1, DT), bconst2(1, DT),
      bconst2(1, DE), bconst2(1, DE),
      bconst2(D, QD), bconst2(D, QD),
      bconst2(DT, QD), bconst2(DT, QD),
      bconst2(DE, QD), bconst2(DE, QD),
      bconst2(QD, D), bconst2(D, D), bconst2(1, D),
      bconst2(D, D), bconst2(1, D),
      bconst2(1, DT), bconst2(1, DT),
  ]
  return pl.pallas_call(
      _tc_body,
      grid=grid,
      in_specs=in_specs,
      out_specs=pl.BlockSpec((BLK, D), lambda i: (i, 0)),
      out_shape=jax.ShapeDtypeStruct((B, D), f32),
  )(src_g, ngh2, ef3, neighbors, edge_deltas, *weights)


def kernel(memory, source_nodes, neighbors, edge_idxs, edge_deltas,
           edge_features, time_w, time_b, Wq, Wk, Wv, Wo,
           fc1_w, fc1_b, fc2_w, fc2_b):
  f32 = jnp.float32

  nbr_flat = neighbors.reshape(-1).astype(jnp.int32)
  eidx_flat = edge_idxs.reshape(-1).astype(jnp.int32)
  sidx = source_nodes.astype(jnp.int32)
  ngh_flat, src_g, ef_flat = _make_sc_gather()(
      memory, nbr_flat, sidx, eidx_flat, edge_features)
  ngh2 = ngh_flat.reshape(B, NN * D)
  ef3 = ef_flat.reshape(B, NN, DE)

  scale = 1.0 / math.sqrt(DH)
  st = jnp.cos(time_b)
  qc = st @ Wq[D:, :]
  ws = []
  for h in range(H):
    hb = slice(h * DH, (h + 1) * DH)
    bfull = (Wq[:D, hb] @ Wk[:, hb].T) * scale
    cu = ((qc[hb] @ Wk[:, hb].T) * scale)[None, :]
    vw = Wv[:, hb] @ Wo[hb, :]
    ws.append((bfull, cu, vw))
  (b0, c0, v0), (b1, c1, v1) = ws
  weights = [
      b0[:, :D], b1[:, :D], b0[:, D:D + DT], b1[:, D:D + DT],
      b0[:, D + DT:], b1[:, D + DT:],
      c0[:, :D], c1[:, :D], c0[:, D:D + DT], c1[:, D:D + DT],
      c0[:, D + DT:], c1[:, D + DT:],
      v0[:D], v1[:D], v0[D:D + DT], v1[D:D + DT], v0[D + DT:], v1[D + DT:],
      fc1_w[:QD], fc1_w[QD:], fc1_b[None, :],
      fc2_w, fc2_b[None, :],
      time_w[None, :], time_b[None, :],
  ]
  weights = [w.astype(f32) for w in weights]

  return _tc_call(src_g, ngh2, ef3, neighbors, edge_deltas.astype(f32),
                  weights)


# speedup vs baseline: 3.2834x; 1.0333x over previous
import functools
import math

import jax
import jax.numpy as jnp
from jax import lax
from jax.experimental import pallas as pl
from jax.experimental.pallas import tpu as pltpu
from jax.experimental.pallas import tpu_sc as plsc

N = 10000
D = 256
DT = 100
DE = 16
B = 4096
NN = 16
NE = 160000
H = 2
QD = D + DT
KD = D + DT + DE
DH = QD // H

BLK = 256


def _make_sc_gather():
  info = plsc.get_sparse_core_info()
  nc, ns = info.num_cores, info.num_subcores
  nw = nc * ns
  ngh_per_w = (B * NN) // nw
  ngh_chunk = 128
  n_chunks = ngh_per_w // ngh_chunk
  src_per_w = B // nw
  ef_per_w = (B * NN) // nw

  mesh = plsc.VectorSubcoreMesh(core_axis_name="c", subcore_axis_name="s")

  @functools.partial(
      pl.kernel,
      mesh=mesh,
      compiler_params=pltpu.CompilerParams(use_tc_tiling_on_sc=False),
      out_type=[
          jax.ShapeDtypeStruct((B * NN, D), jnp.float32),
          jax.ShapeDtypeStruct((B, D), jnp.float32),
          jax.ShapeDtypeStruct((B * NN, DE), jnp.float32),
      ],
      scratch_types=[
          pltpu.VMEM((ngh_chunk,), jnp.int32),
          pltpu.VMEM((ngh_chunk, D), jnp.float32),
          pltpu.VMEM((src_per_w,), jnp.int32),
          pltpu.VMEM((ef_per_w,), jnp.int32),
          pltpu.VMEM((ef_per_w, DE), jnp.float32),
          pltpu.SemaphoreType.DMA,
      ],
  )
  def sc_gather(mem_hbm, nbr_hbm, sidx_hbm, eidx_hbm, ef_hbm,
                ngh_out, src_out, ef_out,
                idx_v, rows_v, sidx_v, eidx_v, efrows_v, sem):
    wid = lax.axis_index("s") * nc + lax.axis_index("c")

    ebase = wid * ef_per_w
    pltpu.sync_copy(eidx_hbm.at[pl.ds(ebase, ef_per_w)], eidx_v)
    pltpu.async_copy(ef_hbm.at[eidx_v], efrows_v, sem).wait()
    pltpu.sync_copy(efrows_v, ef_out.at[pl.ds(ebase, ef_per_w)])

    sbase = wid * src_per_w
    pltpu.sync_copy(sidx_hbm.at[pl.ds(sbase, src_per_w)], sidx_v)
    pltpu.async_copy(mem_hbm.at[sidx_v], rows_v.at[pl.ds(0, src_per_w)],
                     sem).wait()
    pltpu.sync_copy(rows_v.at[pl.ds(0, src_per_w)],
                    src_out.at[pl.ds(sbase, src_per_w)])

    for j in range(n_chunks):
      base = wid * ngh_per_w + j * ngh_chunk
      pltpu.sync_copy(nbr_hbm.at[pl.ds(base, ngh_chunk)], idx_v)
      pltpu.async_copy(mem_hbm.at[idx_v], rows_v, sem).wait()
      pltpu.sync_copy(rows_v, ngh_out.at[pl.ds(base, ngh_chunk)])

  return sc_gather


def _tc_body(src_ref, ngh_ref, ef_ref, nbr_ref, dt_ref,
             bm0_ref, bm1_ref, bt0_ref, bt1_ref, be0_ref, be1_ref,
             cm0_ref, cm1_ref, ct0_ref, ct1_ref, ce0_ref, ce1_ref,
             vm0_ref, vm1_ref, vt0_ref, vt1_ref, ve0_ref, ve1_ref,
             fc1a_ref, fc1b_ref, fc1bias_ref, fc2_ref, fc2bias_ref,
             tw_ref, tb_ref, out_ref):
  f32 = jnp.float32
  src = src_ref[...]
  tw = tw_ref[...]
  tb = tb_ref[...]

  def mm(a, b):
    return jax.lax.dot(a, b, preferred_element_type=f32)

  um0 = mm(src, bm0_ref[...]) + cm0_ref[...]
  um1 = mm(src, bm1_ref[...]) + cm1_ref[...]
  ut0 = mm(src, bt0_ref[...]) + ct0_ref[...]
  ut1 = mm(src, bt1_ref[...]) + ct1_ref[...]
  ue0 = mm(src, be0_ref[...]) + ce0_ref[...]
  ue1 = mm(src, be1_ref[...]) + ce1_ref[...]

  od = jnp.ones((D, 1), f32)
  ot = jnp.ones((DT, 1), f32)
  oe = jnp.ones((DE, 1), f32)

  neg = jnp.float32(-1e10)
  s0l, s1l, ml = [], [], []
  for n in range(NN):
    ngh_n = ngh_ref[:, n * D:(n + 1) * D]
    ef_n = ef_ref[:, n, :]
    dt_n = dt_ref[:, n:n + 1]
    et_n = jnp.cos(dt_n * tw + tb)
    m_n = nbr_ref[:, n:n + 1] == 0
    pm = jnp.concatenate([ngh_n * um0, ngh_n * um1], axis=0)
    pt = jnp.concatenate([et_n * ut0, et_n * ut1], axis=0)
    pe = jnp.concatenate([ef_n * ue0, ef_n * ue1], axis=0)
    s = mm(pm, od) + mm(pt, ot) + mm(pe, oe)
    s0 = s[:BLK]
    s1 = s[BLK:]
    s0l.append(jnp.where(m_n, neg, s0))
    s1l.append(jnp.where(m_n, neg, s1))
    ml.append(m_n)

  mx0 = functools.reduce(jnp.maximum, s0l)
  mx1 = functools.reduce(jnp.maximum, s1l)
  e0l = [jnp.exp(s - mx0) for s in s0l]
  e1l = [jnp.exp(s - mx1) for s in s1l]
  inv0 = 1.0 / functools.reduce(jnp.add, e0l)
  inv1 = 1.0 / functools.reduce(jnp.add, e1l)

  accm0 = jnp.zeros((BLK, D), f32)
  accm1 = jnp.zeros((BLK, D), f32)
  acct0 = jnp.zeros((BLK, DT), f32)
  acct1 = jnp.zeros((BLK, DT), f32)
  acce0 = jnp.zeros((BLK, DE), f32)
  acce1 = jnp.zeros((BLK, DE), f32)
  for n in range(NN):
    ngh_n = ngh_ref[:, n * D:(n + 1) * D]
    ef_n = ef_ref[:, n, :]
    dt_n = dt_ref[:, n:n + 1]
    et_n = jnp.cos(dt_n * tw + tb)
    w0 = e0l[n] * inv0
    w1 = e1l[n] * inv1
    accm0 = accm0 + w0 * ngh_n
    accm1 = accm1 + w1 * ngh_n
    acct0 = acct0 + w0 * et_n
    acct1 = acct1 + w1 * et_n
    acce0 = acce0 + w0 * ef_n
    acce1 = acce1 + w1 * ef_n

  attn_out = (mm(accm0, vm0_ref[...]) + mm(accm1, vm1_ref[...]) +
              mm(acct0, vt0_ref[...]) + mm(acct1, vt1_ref[...]) +
              mm(acce0, ve0_ref[...]) + mm(acce1, ve1_ref[...]))

  all_masked = functools.reduce(jnp.logical_and, ml)
  attn_out = jnp.where(all_masked, 0.0, attn_out)

  h1 = mm(attn_out, fc1a_ref[...]) + mm(src, fc1b_ref[...]) + fc1bias_ref[...]
  h1 = jnp.maximum(h1, 0.0)
  out_ref[...] = mm(h1, fc2_ref[...]) + fc2bias_ref[...]


def _tc_call(src_g, ngh2, ef3, neighbors, edge_deltas, weights):
  f32 = jnp.float32
  grid = (B // BLK,)
  bspec_batch2 = lambda w: pl.BlockSpec((BLK, w), lambda i: (i, 0))
  bconst2 = lambda r, c: pl.BlockSpec((r, c), lambda i: (0, 0))

  in_specs = [
      bspec_batch2(D),
      bspec_batch2(NN * D),
      pl.BlockSpec((BLK, NN, DE), lambda i: (i, 0, 0)),
      bspec_batch2(NN),
      bspec_batch2(NN),
      bconst2(D, D), bconst2(D, D),
      bconst2(D, DT), bconst2(D, DT),
      bconst2(D, DE), bconst2(D, DE),
      bconst2(1, D), bconst2(1, D),
      bconst2(1, DT), bconst2(1, DT),
      bconst2(1, DE), bconst2(1, DE),
      bconst2(D, QD), bconst2(D, QD),
      bconst2(DT, QD), bconst2(DT, QD),
      bconst2(DE, QD), bconst2(DE, QD),
      bconst2(QD, D), bconst2(D, D), bconst2(1, D),
      bconst2(D, D), bconst2(1, D),
      bconst2(1, DT), bconst2(1, DT),
  ]
  return pl.pallas_call(
      _tc_body,
      grid=grid,
      in_specs=in_specs,
      out_specs=pl.BlockSpec((BLK, D), lambda i: (i, 0)),
      out_shape=jax.ShapeDtypeStruct((B, D), f32),
  )(src_g, ngh2, ef3, neighbors, edge_deltas, *weights)


def kernel(memory, source_nodes, neighbors, edge_idxs, edge_deltas,
           edge_features, time_w, time_b, Wq, Wk, Wv, Wo,
           fc1_w, fc1_b, fc2_w, fc2_b):
  f32 = jnp.float32

  nbr_flat = neighbors.reshape(-1).astype(jnp.int32)
  eidx_flat = edge_idxs.reshape(-1).astype(jnp.int32)
  sidx = source_nodes.astype(jnp.int32)
  ngh_flat, src_g, ef_flat = _make_sc_gather()(
      memory, nbr_flat, sidx, eidx_flat, edge_features)
  ngh2 = ngh_flat.reshape(B, NN * D)
  ef3 = ef_flat.reshape(B, NN, DE)

  scale = 1.0 / math.sqrt(DH)
  st = jnp.cos(time_b)
  qc = st @ Wq[D:, :]
  ws = []
  for h in range(H):
    hb = slice(h * DH, (h + 1) * DH)
    bfull = (Wq[:D, hb] @ Wk[:, hb].T) * scale
    cu = ((qc[hb] @ Wk[:, hb].T) * scale)[None, :]
    vw = Wv[:, hb] @ Wo[hb, :]
    ws.append((bfull, cu, vw))
  (b0, c0, v0), (b1, c1, v1) = ws
  weights = [
      b0[:, :D], b1[:, :D], b0[:, D:D + DT], b1[:, D:D + DT],
      b0[:, D + DT:], b1[:, D + DT:],
      c0[:, :D], c1[:, :D], c0[:, D:D + DT], c1[:, D:D + DT],
      c0[:, D + DT:], c1[:, D + DT:],
      v0[:D], v1[:D], v0[D:D + DT], v1[D:D + DT], v0[D + DT:], v1[D + DT:],
      fc1_w[:QD], fc1_w[QD:], fc1_b[None, :],
      fc2_w, fc2_b[None, :],
      time_w[None, :], time_b[None, :],
  ]
  weights = [w.astype(f32) for w in weights]

  return _tc_call(src_g, ngh2, ef3, neighbors, edge_deltas.astype(f32),
                  weights)
